# baseline (speedup 1.0000x reference)
import functools

import jax
import jax.numpy as jnp
from jax import lax
from jax.experimental import pallas as pl
from jax.experimental.pallas import tpu as pltpu
from jax.experimental.pallas import tpu_sc as plsc

B = 128
S = 8
V = 100000
ROWS = B * S
NC, NS, L = 2, 16, 16
NW = NC * NS
RPW = ROWS // NW
WR = RPW * V // 128
CHR = 200
CH2 = CHR * 128
NCH = RPW * V // CH2
AJ = 8
NEG = -3.4e38

_BIG = 2**30


def _argmax_sc_body(logits_hbm, maxv_hbm, idxv_hbm,
                    bufa, bufb, spa, spb, maxb, idxb,
                    hsema, hsemb, ssema, ssemb):
    cid = lax.axis_index("c")
    sid = lax.axis_index("s")
    wid = sid * NC + cid
    row0 = wid * RPW
    wrow0 = wid * WR
    iota = lax.broadcasted_iota(jnp.int32, (L,), 0)
    bufs = (bufa, bufb)
    sps = (spa, spb)
    hsems = (hsema, hsemb)
    ssems = (ssema, ssemb)

    def hs_slice(c):
        return logits_hbm.at[pl.ds(wrow0 + c * CHR, CHR)]

    def sp_slice(p):
        return sps[p].at[pl.ds(sid * CHR, CHR)]

    NSPL = 5
    QR = CHR // NSPL

    def hs_start(c, p):
        for q in range(NSPL):
            pltpu.async_copy(
                logits_hbm.at[pl.ds(wrow0 + c * CHR + q * QR, QR)],
                sps[p].at[pl.ds(sid * CHR + q * QR, QR)], hsems[p])

    def hs_wait(c, p):
        pltpu.make_async_copy(hs_slice(c), sp_slice(p), hsems[p]).wait()

    def st_start(p):
        for q in range(NSPL):
            pltpu.async_copy(sps[p].at[pl.ds(sid * CHR + q * QR, QR)],
                             bufs[p].at[pl.ds(q * QR, QR)], ssems[p])

    def st_wait(p):
        pltpu.make_async_copy(sp_slice(p), bufs[p], ssems[p]).wait()

    def fresh():
        return (tuple(jnp.full((L,), NEG, jnp.float32) for _ in range(AJ)),
                tuple(jnp.zeros((L,), jnp.int32) for _ in range(AJ)))

    def sweep(buf, g, i0, i1, rms, ris):
        def load(i):
            return tuple(buf[i, pl.ds(j * L, L)] for j in range(AJ))

        def body(i, c):
            rms, ris, vs = c
            nxt = load(jnp.minimum(i + 1, CHR - 1))
            gv = jnp.full((L,), g * CHR + i, jnp.int32)
            nm, ni = [], []
            for j in range(AJ):
                m = vs[j] > rms[j]
                nm.append(jnp.where(m, vs[j], rms[j]))
                ni.append(jnp.where(m, gv, ris[j]))
            return tuple(nm), tuple(ni), nxt

        first = load(jnp.minimum(i0, CHR - 1))
        rms, ris, _ = lax.fori_loop(i0, i1, body, (rms, ris, first))
        return rms, ris

    def finalize_store(slot, local_row, rms, ris):
        off0 = -local_row * V
        bm = rms[0]
        bi = ris[0] * 128 + iota + off0
        for j in range(1, AJ):
            b = rms[j]
            ib = ris[j] * 128 + iota + (off0 + j * L)
            m1 = b > bm
            bm = jnp.where(m1, b, bm)
            bi = jnp.where(m1, ib, bi)
            bi = jnp.minimum(bi, jnp.where(b == bm, ib, _BIG))
        maxb[pl.ds(slot * L, L)] = bm
        idxb[pl.ds(slot * L, L)] = bi

    hs_start(0, 0)
    hs_wait(0, 0)
    st_start(0)
    hs_start(1, 1)

    def chunk_step(g, p, carry):
        rms, ris, cur = carry
        hs_wait(g + 1, 1 - p)
        st_start(1 - p)
        st_wait(p)
        hs_start(jnp.minimum(g + 2, NCH - 1), p)
        buf = bufs[p]
        cs = g * CH2
        nb = (cur + 1) * V
        h = jnp.clip(cs + CH2 - nb, 0, 1)
        b = nb - cs
        bi_ = b // 128
        rms, ris = sweep(buf, g, 0, jnp.minimum(bi_, CHR), rms, ris)
        ib_safe = jnp.minimum(bi_, CHR - 1)
        js = (b - bi_ * 128) // L
        gv = jnp.full((L,), g * CHR + ib_safe, jnp.int32)
        frm, fri = fresh()
        vs = [buf[ib_safe, pl.ds(j * L, L)] for j in range(AJ)]

        def gate(v, h_x):
            cap = (2 * h_x - 1).astype(jnp.float32) * 3.4e38
            return jnp.minimum(v, jnp.full((L,), cap))

        nm, ni = [], []
        for j in range(AJ):
            vo = gate(vs[j], h * jnp.clip(js - j, 0, 1))
            mo = vo > rms[j]
            nm.append(jnp.where(mo, vo, rms[j]))
            ni.append(jnp.where(mo, gv, ris[j]))
        rms, ris = tuple(nm), tuple(ni)

        slot = cur * h + RPW * (1 - h)
        finalize_store(slot, cur, rms, ris)

        nm, ni = [], []
        for j in range(AJ):
            rm_j = gate(rms[j], 1 - h)
            ri_j = ris[j] * jnp.full((L,), 1 - h)
            vn = gate(vs[j], h * jnp.clip(j + 1 - js, 0, 1))
            mn = vn > rm_j
            nm.append(jnp.where(mn, vn, rm_j))
            ni.append(jnp.where(mn, gv, ri_j))
        rms, ris = tuple(nm), tuple(ni)
        cur = cur + h
        rms, ris = sweep(buf, g, jnp.minimum(bi_ + 1, CHR), CHR, rms, ris)
        return rms, ris, cur

    def pair_body(t, carry):
        for u in range(2):
            carry = chunk_step(2 * t + u, u, carry)
        return carry

    rms0, ris0 = fresh()
    carry = lax.fori_loop(0, (NCH - 1) // 2, pair_body,
                          (rms0, ris0, jnp.int32(0)))
    rms, ris, cur = carry
    st_wait(0)
    rms, ris = sweep(bufs[0], NCH - 1, 0, CHR, rms, ris)
    finalize_store(RPW - 1, RPW - 1, rms, ris)
    hs_wait(NCH - 1, 1)

    pltpu.sync_copy(maxb.at[pl.ds(0, RPW * L)],
                    maxv_hbm.at[pl.ds(row0 * L, RPW * L)])
    pltpu.sync_copy(idxb.at[pl.ds(0, RPW * L)],
                    idxv_hbm.at[pl.ds(row0 * L, RPW * L)])


_argmax_sc = functools.partial(
    pl.kernel,
    out_type=(
        jax.ShapeDtypeStruct((ROWS * L,), jnp.float32),
        jax.ShapeDtypeStruct((ROWS * L,), jnp.int32),
    ),
    mesh=plsc.VectorSubcoreMesh(core_axis_name="c", subcore_axis_name="s",
                                num_cores=NC, num_subcores=NS),
    scratch_types=[
        pltpu.VMEM((CHR, 128), jnp.float32),
        pltpu.VMEM((CHR, 128), jnp.float32),
        pltpu.VMEM_SHARED((NS * CHR, 128), jnp.float32),
        pltpu.VMEM_SHARED((NS * CHR, 128), jnp.float32),
        pltpu.VMEM(((RPW + 1) * L,), jnp.float32),
        pltpu.VMEM(((RPW + 1) * L,), jnp.int32),
        pltpu.SemaphoreType.DMA,
        pltpu.SemaphoreType.DMA,
        pltpu.SemaphoreType.DMA,
        pltpu.SemaphoreType.DMA,
    ],
)(_argmax_sc_body)


def _fold_body(maxv_ref, idxv_ref, tok_ref):
    maxv = maxv_ref[...]
    idxv = idxv_ref[...]
    vmax = jnp.max(maxv, axis=1, keepdims=True)
    tok_ref[...] = jnp.min(jnp.where(maxv == vmax, idxv, _BIG),
                           axis=1, keepdims=True)


_fold_tc = pl.pallas_call(
    _fold_body,
    out_shape=jax.ShapeDtypeStruct((ROWS, 1), jnp.int32),
)


def _finish_body(tok_ref, draft_ref, bonus_ref, out_ref, nrej_ref, last_ref):
    tok = tok_ref[...]
    dr = draft_ref[...]
    bo = bonus_ref[...]
    io = lax.broadcasted_iota(jnp.int32, (B, S), 1)
    m = tok == dr
    n = jnp.min(jnp.where(m, S, io), axis=1, keepdims=True)
    out_ref[:, :S] = jnp.where(io <= n, tok, -1)
    out_ref[:, S:] = jnp.where(n == S, bo, -1)
    nrej_ref[...] = S - n
    lastt = jnp.sum(jnp.where(io == n, tok, 0), axis=1, keepdims=True)
    last_ref[...] = jnp.where(n == S, bo, lastt)


_finish_tc = pl.pallas_call(
    _finish_body,
    out_shape=[
        jax.ShapeDtypeStruct((B, S + 1), jnp.int32),
        jax.ShapeDtypeStruct((B, 1), jnp.int32),
        jax.ShapeDtypeStruct((B, 1), jnp.int32),
    ],
)


def kernel(target_logits, draft_token_ids, bonus_token_ids):
    logits2d = target_logits.reshape(ROWS * V // 128, 128)
    maxv, idxv = _argmax_sc(logits2d)
    tok = _fold_tc(maxv.reshape(ROWS, L), idxv.reshape(ROWS, L))
    out, nrej, last = _finish_tc(tok.reshape(B, S),
                                 draft_token_ids, bonus_token_ids)
    return out, nrej.reshape(B), last.reshape(B)


# speedup vs baseline: 1.0469x; 1.0469x over previous
import functools

import jax
import jax.numpy as jnp
from jax import lax
from jax.experimental import pallas as pl
from jax.experimental.pallas import tpu as pltpu
from jax.experimental.pallas import tpu_sc as plsc

B = 128
S = 8
V = 100000
ROWS = B * S
NC, NS, L = 2, 16, 16
NW = NC * NS
RPW = ROWS // NW
NCHUNK = 5
CH = V // NCHUNK
A = 10
ITERS = CH // (A * L)

_BIG = 2**30


def _argmax_sc_body(logits_hbm, maxv_hbm, idxv_hbm,
                    bufa, bufb, maxb, idxb, sema, semb):
    wid = lax.axis_index("s") * NC + lax.axis_index("c")
    row0 = wid * RPW
    iota = lax.broadcasted_iota(jnp.int32, (L,), 0)
    bufs = (bufa, bufb)
    sems = (sema, semb)

    def start(r, k, which):
        pltpu.async_copy(logits_hbm.at[pl.ds(r * V + k * CH, CH)],
                         bufs[which], sems[which])

    def wait(r, k, which):
        pltpu.make_async_copy(logits_hbm.at[pl.ds(r * V + k * CH, CH)],
                              bufs[which], sems[which]).wait()

    def run_chunk(buf, it0, rms, ris):
        def ibody(i, carry):
            rms, ris = carry
            itv = jnp.full((L,), it0 + i, jnp.int32)
            base = i * (A * L)
            nm, ni = [], []
            for j in range(A):
                v = buf[pl.ds(base + j * L, L)]
                m = v > rms[j]
                nm.append(jnp.maximum(rms[j], v))
                ni.append(jnp.where(m, itv, ris[j]))
            return tuple(nm), tuple(ni)

        return lax.fori_loop(0, ITERS, ibody, (rms, ris))

    def finalize(rr, rms, ris):
        bm = rms[0]
        bi = ris[0] * (A * L) + iota
        for j in range(1, A):
            b = rms[j]
            ib = ris[j] * (A * L) + (j * L) + iota
            take = (b > bm) | ((b == bm) & (ib < bi))
            bm = jnp.where(take, b, bm)
            bi = jnp.where(take, ib, bi)
        maxb[pl.ds((rr - row0) * L, L)] = bm
        idxb[pl.ds((rr - row0) * L, L)] = bi

    start(row0, 0, 0)

    def pair_body(t, unused):
        r_even = row0 + 2 * t
        rms = ris = None
        for u in range(2 * NCHUNK):
            rr = r_even + (u // NCHUNK)
            k = u % NCHUNK
            if u < 2 * NCHUNK - 1:
                rn = r_even + ((u + 1) // NCHUNK)
                kn = (u + 1) % NCHUNK
            else:
                rn = jnp.minimum(r_even + 2, row0 + RPW - 1)
                kn = 0
            start(rn, kn, (u + 1) % 2)
            wait(rr, k, u % 2)
            if k == 0:
                rms = tuple(jnp.full((L,), -3.4e38, jnp.float32)
                            for _ in range(A))
                ris = tuple(jnp.zeros((L,), jnp.int32) for _ in range(A))
            rms, ris = run_chunk(bufs[u % 2], k * ITERS, rms, ris)
            if k == NCHUNK - 1:
                finalize(rr, rms, ris)
        return unused

    lax.fori_loop(0, RPW // 2, pair_body, 0)
    wait(row0 + RPW - 1, 0, 0)
    pltpu.sync_copy(maxb, maxv_hbm.at[pl.ds(row0 * L, RPW * L)])
    pltpu.sync_copy(idxb, idxv_hbm.at[pl.ds(row0 * L, RPW * L)])


_argmax_sc = functools.partial(
    pl.kernel,
    out_type=(
        jax.ShapeDtypeStruct((ROWS * L,), jnp.float32),
        jax.ShapeDtypeStruct((ROWS * L,), jnp.int32),
    ),
    mesh=plsc.VectorSubcoreMesh(core_axis_name="c", subcore_axis_name="s",
                                num_cores=NC, num_subcores=NS),
    scratch_types=[
        pltpu.VMEM((CH,), jnp.float32),
        pltpu.VMEM((CH,), jnp.float32),
        pltpu.VMEM((RPW * L,), jnp.float32),
        pltpu.VMEM((RPW * L,), jnp.int32),
        pltpu.SemaphoreType.DMA,
        pltpu.SemaphoreType.DMA,
    ],
)(_argmax_sc_body)


def _fold_body(maxv_ref, idxv_ref, tok_ref):
    maxv = maxv_ref[...]
    idxv = idxv_ref[...]
    vmax = jnp.max(maxv, axis=1, keepdims=True)
    tok_ref[...] = jnp.min(jnp.where(maxv == vmax, idxv, _BIG),
                           axis=1, keepdims=True)


_fold_tc = pl.pallas_call(
    _fold_body,
    out_shape=jax.ShapeDtypeStruct((ROWS, 1), jnp.int32),
)


def _finish_body(tok_ref, draft_ref, bonus_ref, out_ref, nrej_ref, last_ref):
    tok = tok_ref[...]
    dr = draft_ref[...]
    bo = bonus_ref[...]
    io = lax.broadcasted_iota(jnp.int32, (B, S), 1)
    m = tok == dr
    n = jnp.min(jnp.where(m, S, io), axis=1, keepdims=True)
    out_ref[:, :S] = jnp.where(io <= n, tok, -1)
    out_ref[:, S:] = jnp.where(n == S, bo, -1)
    nrej_ref[...] = S - n
    lastt = jnp.sum(jnp.where(io == n, tok, 0), axis=1, keepdims=True)
    last_ref[...] = jnp.where(n == S, bo, lastt)


_finish_tc = pl.pallas_call(
    _finish_body,
    out_shape=[
        jax.ShapeDtypeStruct((B, S + 1), jnp.int32),
        jax.ShapeDtypeStruct((B, 1), jnp.int32),
        jax.ShapeDtypeStruct((B, 1), jnp.int32),
    ],
)


def kernel(target_logits, draft_token_ids, bonus_token_ids):
    flat = target_logits.reshape(ROWS * V)
    maxv, idxv = _argmax_sc(flat)
    tok = _fold_tc(maxv.reshape(ROWS, L), idxv.reshape(ROWS, L))
    out, nrej, last = _finish_tc(tok.reshape(B, S),
                                 draft_token_ids, bonus_token_ids)
    return out, nrej.reshape(B), last.reshape(B)
